# baseline (device time: 52283 ns/iter reference)
import jax
import jax.numpy as jnp
from jax import lax
from jax.experimental import pallas as pl
from jax.experimental.pallas import tpu as pltpu

VOCAB_PER_X = 8192
LOG2_MAX = 11


def kernel(ids, E):
    t_tokens = ids.shape[0]
    d = E.shape[1]

    def body(count_ref, tok_ref, row_ref, e_ref, out_ref, gsem, ssem, rsem):
        my_x = lax.axis_index("x")
        my_y = lax.axis_index("y")
        my_z = lax.axis_index("z")
        peer = (1 - my_x, my_y, my_z)
        count = count_ref[0]
        n_recv = t_tokens - count

        def remote_desc(src, dst):
            return pltpu.make_async_remote_copy(
                src_ref=src,
                dst_ref=dst,
                send_sem=ssem,
                recv_sem=rsem,
                device_id=peer,
                device_id_type=pl.DeviceIdType.MESH,
            )

        BLK = 8

        def blk_step(b, c):
            p0 = b * BLK
            for k in range(BLK):
                p = p0 + k

                @pl.when(p < count)
                def _():
                    t = tok_ref[p]
                    r = row_ref[p]
                    pltpu.make_async_copy(
                        e_ref.at[r], out_ref.at[t], gsem
                    ).start()
                    remote_desc(e_ref.at[r], out_ref.at[t]).start()

            return c

        lax.fori_loop(0, t_tokens // BLK, blk_step, 0)

        def bulk(n, wait_one):
            for b in range(LOG2_MAX - 1, -1, -1):
                sz = 1 << b

                @pl.when((n & sz) != 0)
                def _():
                    wait_one(sz)

        bulk(count, lambda sz: pltpu.make_async_copy(
            e_ref.at[pl.ds(0, sz), :], out_ref.at[pl.ds(0, sz), :], gsem
        ).wait())
        bulk(count, lambda sz: remote_desc(
            e_ref.at[pl.ds(0, sz), :], out_ref.at[pl.ds(0, sz), :]
        ).wait_send())
        bulk(n_recv, lambda sz: remote_desc(
            e_ref.at[pl.ds(0, sz), :], out_ref.at[pl.ds(0, sz), :]
        ).wait_recv())

    my_x = lax.axis_index("x")
    base = my_x * VOCAB_PER_X
    idx = ids.astype(jnp.int32) - base
    owned = (idx >= 0) & (idx < VOCAB_PER_X)
    pos = jnp.cumsum(owned.astype(jnp.int32)) - 1
    scatter_to = jnp.where(owned, pos, t_tokens)
    iota = jnp.arange(t_tokens, dtype=jnp.int32)
    tok = jnp.zeros((t_tokens,), jnp.int32).at[scatter_to].set(iota, mode="drop")
    row = jnp.zeros((t_tokens,), jnp.int32).at[scatter_to].set(idx, mode="drop")
    count = jnp.sum(owned.astype(jnp.int32)).reshape((1,))

    return pl.pallas_call(
        body,
        out_shape=jax.ShapeDtypeStruct((t_tokens, d), jnp.float32),
        in_specs=[
            pl.BlockSpec(memory_space=pltpu.SMEM),
            pl.BlockSpec(memory_space=pltpu.SMEM),
            pl.BlockSpec(memory_space=pltpu.SMEM),
            pl.BlockSpec(memory_space=pltpu.HBM),
        ],
        out_specs=pl.BlockSpec(memory_space=pltpu.VMEM),
        scratch_shapes=[
            pltpu.SemaphoreType.DMA,
            pltpu.SemaphoreType.DMA,
            pltpu.SemaphoreType.DMA,
        ],
    )(count, tok, row, E)


# device time: 44470 ns/iter; 1.1757x vs baseline; 1.1757x over previous
import jax
import jax.numpy as jnp
from jax import lax
from jax.experimental import pallas as pl
from jax.experimental.pallas import tpu as pltpu

VOCAB_PER_X = 8192
LOG2_MAX = 11


def kernel(ids, E):
    t_tokens = ids.shape[0]
    d = E.shape[1]

    def body(count_ref, pak_ref, e_ref, out_ref, gsem, ssem, rsem):
        my_x = lax.axis_index("x")
        my_y = lax.axis_index("y")
        my_z = lax.axis_index("z")
        peer = ((1 - my_x) * 4 + my_y) * 4 + my_z
        count = count_ref[0]
        n_recv = t_tokens - count

        def remote_desc(src, dst):
            return pltpu.make_async_remote_copy(
                src_ref=src,
                dst_ref=dst,
                send_sem=ssem,
                recv_sem=rsem,
                device_id=peer,
                device_id_type=pl.DeviceIdType.LOGICAL,
            )

        def step(p, c):
            v = pak_ref[p]
            t = v >> 14
            r = v & 16383
            pltpu.make_async_copy(e_ref.at[r], out_ref.at[t], gsem).start()
            remote_desc(e_ref.at[r], out_ref.at[t]).start()
            return c

        lax.fori_loop(0, count, step, 0)

        def bulk(n, wait_one):
            for b in range(LOG2_MAX - 1, -1, -1):
                sz = 1 << b

                @pl.when((n & sz) != 0)
                def _():
                    wait_one(sz)

        bulk(count, lambda sz: pltpu.make_async_copy(
            e_ref.at[pl.ds(0, sz), :], out_ref.at[pl.ds(0, sz), :], gsem
        ).wait())
        bulk(count, lambda sz: remote_desc(
            e_ref.at[pl.ds(0, sz), :], out_ref.at[pl.ds(0, sz), :]
        ).wait_send())
        bulk(n_recv, lambda sz: remote_desc(
            e_ref.at[pl.ds(0, sz), :], out_ref.at[pl.ds(0, sz), :]
        ).wait_recv())

    my_x = lax.axis_index("x")
    base = my_x * VOCAB_PER_X
    idx = ids.astype(jnp.int32) - base
    owned = (idx >= 0) & (idx < VOCAB_PER_X)
    pos = jnp.cumsum(owned.astype(jnp.int32)) - 1
    scatter_to = jnp.where(owned, pos, t_tokens)
    iota = jnp.arange(t_tokens, dtype=jnp.int32)
    packed = (iota << 14) | jnp.where(owned, idx, 0)
    pak = jnp.zeros((t_tokens,), jnp.int32).at[scatter_to].set(packed, mode="drop")
    count = jnp.sum(owned.astype(jnp.int32)).reshape((1,))

    return pl.pallas_call(
        body,
        out_shape=jax.ShapeDtypeStruct((t_tokens, d), jnp.float32),
        in_specs=[
            pl.BlockSpec(memory_space=pltpu.SMEM),
            pl.BlockSpec(memory_space=pltpu.SMEM),
            pl.BlockSpec(memory_space=pltpu.HBM),
        ],
        out_specs=pl.BlockSpec(memory_space=pltpu.VMEM),
        scratch_shapes=[
            pltpu.SemaphoreType.DMA,
            pltpu.SemaphoreType.DMA,
            pltpu.SemaphoreType.DMA,
        ],
    )(count, pak, E)
